# Initial kernel scaffold; baseline (speedup 1.0000x reference)
import jax
import jax.numpy as jnp
from jax.experimental import pallas as pl


def kernel(x, params):
    raise NotImplementedError("write your pallas kernel here")


# speedup vs baseline: 5.3243x; 5.3243x over previous
import functools

import jax
import jax.numpy as jnp
import numpy as np
from jax import lax
from jax.experimental import pallas as pl
from jax.experimental.pallas import tpu as pltpu
from jax.experimental.pallas import tpu_sc as plsc

_NSAMPLE = 50
_SC_WORKERS = 32


def _rup(a, m):
    return (a + m - 1) // m * m


def _bf(x):
    return x.astype(jnp.bfloat16)


def _fps_body(xyz_ref, out_ref, *, npoint, n):
    xt = xyz_ref[0]
    iota_n = lax.broadcasted_iota(jnp.int32, (1, n), 1)
    iota_s = lax.broadcasted_iota(jnp.int32, (1, npoint), 1)

    def step(s, carry):
        dist, far, acc = carry
        oh = (iota_n == far).astype(jnp.float32)
        c = jnp.sum(xt * oh, axis=1, keepdims=True)
        acc = acc + c * (iota_s == s).astype(jnp.float32)
        d = xt - c
        d = d * d
        dsum = d[1:2] + d[2:3] + d[0:1]
        dist = jnp.minimum(dist, dsum)
        m = jnp.max(dist)
        far2 = jnp.min(jnp.where(dist == m, iota_n, n))
        return dist, far2, acc

    dist0 = jnp.full((1, n), 1e10, jnp.float32)
    acc0 = jnp.zeros((3, npoint), jnp.float32)
    _, _, acc = lax.fori_loop(0, npoint, step, (dist0, jnp.int32(0), acc0))
    out_ref[0] = acc


def _fps(xyz_t, npoint):
    b, _, n = xyz_t.shape
    return pl.pallas_call(
        functools.partial(_fps_body, npoint=npoint, n=n),
        grid=(b,),
        in_specs=[pl.BlockSpec((1, 3, n), lambda i: (i, 0, 0))],
        out_specs=pl.BlockSpec((1, 3, npoint), lambda i: (i, 0, 0)),
        out_shape=jax.ShapeDtypeStruct((b, 3, npoint), jnp.float32),
    )(xyz_t)


def _ballq_body(xyz_ref, q_ref, out_ref, *, n, st, r2, nsample, xn_order,
                qn_order):
    xt = xyz_ref[0]
    q = q_ref[0]
    xx = xt * xt
    o = xn_order
    xn = xx[o[0]:o[0] + 1] + xx[o[1]:o[1] + 1] + xx[o[2]:o[2] + 1]
    qq = q * q
    o = qn_order
    qn = (qq[:, o[0]:o[0] + 1] + qq[:, o[1]:o[1] + 1]
          + qq[:, o[2]:o[2] + 1])
    cross = jnp.dot(_bf(q), _bf(xt), preferred_element_type=jnp.float32)
    d = qn + xn - 2.0 * cross
    iota = lax.broadcasted_iota(jnp.int32, (st, n), 1)
    iota_k = lax.broadcasted_iota(jnp.int32, (st, nsample), 1)
    mask = d <= r2
    a = jnp.where(mask, iota, n)
    first = jnp.min(a, axis=1, keepdims=True)
    firstc = jnp.where(first >= n, n - 1, first)
    out0 = jnp.broadcast_to(firstc, (st, nsample))
    a = jnp.where(a == first, n, a)
    cnt = jnp.sum(mask.astype(jnp.int32), axis=1, keepdims=True)
    kmax = jnp.minimum(jnp.max(cnt), nsample)

    def cond(carry):
        k, _, _ = carry
        return k < kmax

    def body(carry):
        k, a, out = carry
        v = jnp.min(a, axis=1, keepdims=True)
        veff = jnp.where(v >= n, firstc, v)
        out = jnp.where(iota_k == k, veff, out)
        a = jnp.where(a == v, n, a)
        return k + 1, a, out

    _, _, out = lax.while_loop(cond, body, (jnp.int32(1), a, out0))
    out_ref[0] = out


def _ballq(xyz_t, new_xyz, radius, st, xn_order=(1, 2, 0), qn_order=(0, 1, 2)):
    b, _, n = xyz_t.shape
    s = new_xyz.shape[1]
    r2 = np.float32(radius ** 2)
    return pl.pallas_call(
        functools.partial(_ballq_body, n=n, st=st, r2=r2, nsample=_NSAMPLE,
                          xn_order=xn_order, qn_order=qn_order),
        grid=(b, s // st),
        in_specs=[
            pl.BlockSpec((1, 3, n), lambda i, j: (i, 0, 0)),
            pl.BlockSpec((1, st, 3), lambda i, j: (i, j, 0)),
        ],
        out_specs=pl.BlockSpec((1, st, _NSAMPLE), lambda i, j: (i, j, 0)),
        out_shape=jax.ShapeDtypeStruct((b, s, _NSAMPLE), jnp.int32),
    )(xyz_t, new_xyz)


def _pick_chunk(bw):
    for c in range(128, 7, -8):
        if bw % c == 0:
            return c
    return 8


def _sc_gather_call(table, idx, v, d, bp):
    bw = bp // _SC_WORKERS
    ch = _pick_chunk(bw)
    nch = bw // ch
    mesh = plsc.VectorSubcoreMesh(core_axis_name="c", subcore_axis_name="s")

    @functools.partial(
        pl.kernel,
        mesh=mesh,
        compiler_params=pltpu.CompilerParams(use_tc_tiling_on_sc=False),
        out_type=jax.ShapeDtypeStruct((bp, d), jnp.float32),
        scratch_types=[
            pltpu.VMEM((bw,), jnp.int32),
            pltpu.VMEM((bw, d), jnp.float32),
            pltpu.SemaphoreType.DMA,
        ],
    )
    def k(table_hbm, idx_hbm, out_hbm, idx_v, rows_v, sem):
        wid = lax.axis_index("s") * 2 + lax.axis_index("c")
        base = wid * bw
        pltpu.sync_copy(idx_hbm.at[pl.ds(base, bw)], idx_v)
        for g0 in range(0, nch, 8):
            cps = [
                pltpu.async_copy(
                    table_hbm.at[idx_v.at[pl.ds(c * ch, ch)]],
                    rows_v.at[pl.ds(c * ch, ch)],
                    sem,
                )
                for c in range(g0, min(g0 + 8, nch))
            ]
            for cp in cps:
                cp.wait()
        pltpu.sync_copy(rows_v, out_hbm.at[pl.ds(base, bw)])

    return k(table, idx)


def _gather_rows(table, flat_idx):
    v, d = table.shape
    return _sc_gather_call(table, flat_idx, v, d, flat_idx.shape[0])


def _prologue(x, mu, gm, sv, bt):
    return jnp.maximum(gm * (x - mu) / sv + bt, 0.0)


def _sa_l1_body(g_ref, cen_ref, wt_ref, mu_ref, gm_ref, sv_ref, bt_ref,
                y_ref, *, mt, st, has_feat):
    g = g_ref[0]
    cen = cen_ref[0]
    x3 = g[:, :3].reshape(st, _NSAMPLE, 3) - cen[:, None, :]
    x3 = x3.reshape(mt, 3)
    xf = g[:, 3:]
    if has_feat:
        xf = _prologue(xf, mu_ref[0, 3:][None, :], gm_ref[0, 3:][None, :],
                       sv_ref[0, 3:][None, :], bt_ref[0, 3:][None, :])
    x = jnp.concatenate([x3, xf], axis=1)
    y = jnp.dot(_bf(x), _bf(wt_ref[...]), preferred_element_type=jnp.float32)
    y_ref[0] = y


def _sa_l1(g, cen, wt, aff, mt, has_feat):
    b, m, d16 = g.shape
    st = mt // _NSAMPLE
    co = wt.shape[1]
    return pl.pallas_call(
        functools.partial(_sa_l1_body, mt=mt, st=st, has_feat=has_feat),
        grid=(b, m // mt),
        in_specs=[
            pl.BlockSpec((1, mt, d16), lambda i, j: (i, j, 0)),
            pl.BlockSpec((1, st, 3), lambda i, j: (i, j, 0)),
            pl.BlockSpec((d16, co), lambda i, j: (0, 0)),
            pl.BlockSpec((1, d16), lambda i, j: (0, 0)),
            pl.BlockSpec((1, d16), lambda i, j: (0, 0)),
            pl.BlockSpec((1, d16), lambda i, j: (0, 0)),
            pl.BlockSpec((1, d16), lambda i, j: (0, 0)),
        ],
        out_specs=pl.BlockSpec((1, mt, co), lambda i, j: (i, j, 0)),
        out_shape=jax.ShapeDtypeStruct((b, m, co), jnp.float32),
    )(g, cen, wt, *aff)


def _mm_body(x_ref, wt_ref, mu_ref, gm_ref, sv_ref, bt_ref, y_ref):
    x = _prologue(x_ref[0], mu_ref[...], gm_ref[...], sv_ref[...], bt_ref[...])
    y = jnp.dot(_bf(x), _bf(wt_ref[...]), preferred_element_type=jnp.float32)
    y_ref[0] = y


def _mm(x, wt, aff, mt):
    b, m, ci = x.shape
    co = wt.shape[1]
    return pl.pallas_call(
        _mm_body,
        grid=(b, m // mt),
        in_specs=[
            pl.BlockSpec((1, mt, ci), lambda i, j: (i, j, 0)),
            pl.BlockSpec((ci, co), lambda i, j: (0, 0)),
            pl.BlockSpec((1, ci), lambda i, j: (0, 0)),
            pl.BlockSpec((1, ci), lambda i, j: (0, 0)),
            pl.BlockSpec((1, ci), lambda i, j: (0, 0)),
            pl.BlockSpec((1, ci), lambda i, j: (0, 0)),
        ],
        out_specs=pl.BlockSpec((1, mt, co), lambda i, j: (i, j, 0)),
        out_shape=jax.ShapeDtypeStruct((b, m, co), jnp.float32),
    )(x, wt, *aff)


def _mm_pool_body(x_ref, wt_ref, mu_ref, gm_ref, sv_ref, bt_ref,
                  y_ref, p_ref, *, mt, st):
    x = _prologue(x_ref[0], mu_ref[...], gm_ref[...], sv_ref[...], bt_ref[...])
    y = jnp.dot(_bf(x), _bf(wt_ref[...]), preferred_element_type=jnp.float32)
    y_ref[0] = y
    co = y.shape[1]
    p_ref[0] = jnp.max(y.reshape(st, _NSAMPLE, co), axis=1)


def _mm_pool(x, wt, aff, mt):
    b, m, ci = x.shape
    co = wt.shape[1]
    st = mt // _NSAMPLE
    s = m // _NSAMPLE
    return pl.pallas_call(
        functools.partial(_mm_pool_body, mt=mt, st=st),
        grid=(b, m // mt),
        in_specs=[
            pl.BlockSpec((1, mt, ci), lambda i, j: (i, j, 0)),
            pl.BlockSpec((ci, co), lambda i, j: (0, 0)),
            pl.BlockSpec((1, ci), lambda i, j: (0, 0)),
            pl.BlockSpec((1, ci), lambda i, j: (0, 0)),
            pl.BlockSpec((1, ci), lambda i, j: (0, 0)),
            pl.BlockSpec((1, ci), lambda i, j: (0, 0)),
        ],
        out_specs=[
            pl.BlockSpec((1, mt, co), lambda i, j: (i, j, 0)),
            pl.BlockSpec((1, st, co), lambda i, j: (i, j, 0)),
        ],
        out_shape=[
            jax.ShapeDtypeStruct((b, m, co), jnp.float32),
            jax.ShapeDtypeStruct((b, s, co), jnp.float32),
        ],
    )(x, wt, *aff)


def _fp_l1_body(x1_ref, x2_ref, p1_ref, p2_ref, wt_ref,
                mu1_ref, gm1_ref, sv1_ref, bt1_ref,
                mu2_ref, gm2_ref, sv2_ref, bt2_ref,
                y_ref, *, nt, s2, p1_relu):
    x1 = x1_ref[0]
    x2 = x2_ref[0]
    x1s = x1 * x1
    n1 = x1s[:, 0:1] + x1s[:, 1:2] + x1s[:, 2:3]
    x2s = x2 * x2
    n2 = x2s[:, 0:1] + x2s[:, 1:2] + x2s[:, 2:3]
    cross = lax.dot_general(
        _bf(x1), _bf(x2), (((1,), (1,)), ((), ())),
        preferred_element_type=jnp.float32)
    d = n1 + n2.T - 2.0 * cross
    iota = lax.broadcasted_iota(jnp.int32, (nt, s2), 1)
    p2 = _prologue(p2_ref[0], mu2_ref[...], gm2_ref[...], sv2_ref[...],
                   bt2_ref[...])
    dd = d
    vs, rows = [], []
    for _ in range(3):
        v = jnp.min(dd, axis=1, keepdims=True)
        am = jnp.min(jnp.where(dd == v, iota, s2), axis=1, keepdims=True)
        sel = (iota == am).astype(jnp.float32)
        dd = jnp.where(iota == am, jnp.float32(1e30), dd)
        vs.append(v)
        rows.append(jnp.dot(sel, p2, preferred_element_type=jnp.float32))
    r = [1.0 / (v + 1e-8) for v in vs]
    rs = r[0] + r[1] + r[2]
    w = [ri / rs for ri in r]
    interp = rows[0] * w[0] + rows[1] * w[1] + rows[2] * w[2]
    p1 = p1_ref[0]
    if p1_relu:
        p1 = _prologue(p1, mu1_ref[...], gm1_ref[...], sv1_ref[...],
                       bt1_ref[...])
    x = jnp.concatenate([p1, interp], axis=1)
    y = jnp.dot(_bf(x), _bf(wt_ref[...]), preferred_element_type=jnp.float32)
    y_ref[0] = y


def _fp_l1(x1, x2, p1, p2, wt, aff1, aff2, nt, p1_relu):
    b, n, _ = x1.shape
    s2 = x2.shape[1]
    c1 = p1.shape[2]
    c2 = p2.shape[2]
    co = wt.shape[1]
    return pl.pallas_call(
        functools.partial(_fp_l1_body, nt=nt, s2=s2, p1_relu=p1_relu),
        grid=(b, n // nt),
        in_specs=[
            pl.BlockSpec((1, nt, 3), lambda i, j: (i, j, 0)),
            pl.BlockSpec((1, s2, 3), lambda i, j: (i, 0, 0)),
            pl.BlockSpec((1, nt, c1), lambda i, j: (i, j, 0)),
            pl.BlockSpec((1, s2, c2), lambda i, j: (i, 0, 0)),
            pl.BlockSpec((c1 + c2, co), lambda i, j: (0, 0)),
            pl.BlockSpec((1, c1), lambda i, j: (0, 0)),
            pl.BlockSpec((1, c1), lambda i, j: (0, 0)),
            pl.BlockSpec((1, c1), lambda i, j: (0, 0)),
            pl.BlockSpec((1, c1), lambda i, j: (0, 0)),
            pl.BlockSpec((1, c2), lambda i, j: (0, 0)),
            pl.BlockSpec((1, c2), lambda i, j: (0, 0)),
            pl.BlockSpec((1, c2), lambda i, j: (0, 0)),
            pl.BlockSpec((1, c2), lambda i, j: (0, 0)),
        ],
        out_specs=pl.BlockSpec((1, nt, co), lambda i, j: (i, j, 0)),
        out_shape=jax.ShapeDtypeStruct((b, n, co), jnp.float32),
    )(x1, x2, p1, p2, wt, *aff1, *aff2)


def _head2_body(x_ref, wt_ref, b_ref, mu_ref, gm_ref, sv_ref, bt_ref, out_ref):
    x = _prologue(x_ref[0], mu_ref[...], gm_ref[...], sv_ref[...], bt_ref[...])
    y = jnp.dot(_bf(x), _bf(wt_ref[...]), preferred_element_type=jnp.float32)
    y = y + b_ref[...]
    m = jnp.max(y, axis=1, keepdims=True)
    ym = y - m
    lse = jnp.log(jnp.sum(jnp.exp(ym), axis=1, keepdims=True))
    out_ref[0] = ym - lse


def _head2(x, wt, bias, aff, mt):
    b, m, ci = x.shape
    co = wt.shape[1]
    return pl.pallas_call(
        _head2_body,
        grid=(b, m // mt),
        in_specs=[
            pl.BlockSpec((1, mt, ci), lambda i, j: (i, j, 0)),
            pl.BlockSpec((ci, co), lambda i, j: (0, 0)),
            pl.BlockSpec((1, co), lambda i, j: (0, 0)),
            pl.BlockSpec((1, ci), lambda i, j: (0, 0)),
            pl.BlockSpec((1, ci), lambda i, j: (0, 0)),
            pl.BlockSpec((1, ci), lambda i, j: (0, 0)),
            pl.BlockSpec((1, ci), lambda i, j: (0, 0)),
        ],
        out_specs=pl.BlockSpec((1, mt, co), lambda i, j: (i, j, 0)),
        out_shape=jax.ShapeDtypeStruct((b, m, co), jnp.float32),
    )(x, wt, bias, *aff)


def _bn_aff(y, shape4d, gamma, beta):
    y4 = y.reshape(shape4d)
    axes = tuple(range(len(shape4d) - 1))
    mean = jnp.mean(y4, axis=axes)
    var = jnp.var(y4, axis=axes)
    sv = jnp.sqrt(var + 1e-5)
    return mean[None, :], gamma[None, :], sv[None, :], beta[None, :]


def _pad_aff(aff, c, d16):
    mu, gm, sv, bt = aff
    z3 = jnp.zeros((1, 3), jnp.float32)
    zp = jnp.zeros((1, d16 - 3 - c), jnp.float32)
    op = jnp.ones((1, d16 - 3 - c), jnp.float32)
    return (
        jnp.concatenate([z3, mu, zp], axis=1),
        jnp.concatenate([z3, gm, zp], axis=1),
        jnp.concatenate([jnp.ones((1, 3), jnp.float32), sv, op], axis=1),
        jnp.concatenate([z3, bt, zp], axis=1),
    )


def _prep_w(w):
    return jnp.transpose(w)


def _sa_stage(xyz, xyz_t, feat_raw, feat_aff, npoint, radius, params,
              st_ballq, mt):
    b, n, _ = xyz.shape
    new_xyz_t = _fps(xyz_t, npoint)
    new_xyz = jnp.transpose(new_xyz_t, (0, 2, 1))
    idx = _ballq(xyz_t, new_xyz, radius, st_ballq)

    if feat_raw is None:
        c = 0
        table = xyz
    else:
        c = feat_raw.shape[2]
        table = jnp.concatenate([xyz, feat_raw], axis=2)
    d16 = _rup(3 + c, 16)
    table = jnp.pad(table, ((0, 0), (0, 0), (0, d16 - 3 - c)))
    table = table.reshape(b * n, d16)
    flat = (idx + (jnp.arange(b, dtype=jnp.int32) * n)[:, None, None]).reshape(-1)
    bp = _rup(flat.shape[0], 256)
    flat = jnp.pad(flat, (0, bp - flat.shape[0]))
    g = _gather_rows(table, flat)[: b * npoint * _NSAMPLE]
    g = g.reshape(b, npoint * _NSAMPLE, d16)

    (w1, b1, g1, be1), (w2, b2, g2, be2), (w3, b3, g3, be3) = params
    wt1 = jnp.pad(_prep_w(w1), ((0, d16 - 3 - c), (0, 0)))
    if feat_raw is None:
        zero = jnp.zeros((1, d16), jnp.float32)
        aff_in = (zero, zero, jnp.ones((1, d16), jnp.float32), zero)
    else:
        aff_in = _pad_aff(feat_aff, c, d16)
    sh4 = (b, npoint, _NSAMPLE, -1)
    y1 = _sa_l1(g, new_xyz, wt1, aff_in, mt, feat_raw is not None)
    y1 = y1 + b1[None, None, :]
    a1 = _bn_aff(y1, sh4, g1, be1)
    y2 = _mm(y1, _prep_w(w2), a1, mt)
    y2 = y2 + b2[None, None, :]
    a2 = _bn_aff(y2, sh4, g2, be2)
    y3, pooled = _mm_pool(y2, _prep_w(w3), a2, mt)
    y3 = y3 + b3[None, None, :]
    pooled = pooled + b3[None, None, :]
    a3 = _bn_aff(y3, sh4, g3, be3)
    return new_xyz, new_xyz_t, pooled, a3


def _fp_stage(xyz1, xyz2, p1, p1_aff, p1_relu, p2, p2_aff, params, nt, mt):
    b, n, _ = xyz1.shape
    c1 = p1.shape[2]
    w, bb, gg, be = params[0]
    if p1_aff is None:
        z = jnp.zeros((1, c1), jnp.float32)
        p1_aff = (z, z, jnp.ones((1, c1), jnp.float32), z)
    sh3 = (b, n, -1)
    y = _fp_l1(xyz1, xyz2, p1, p2, _prep_w(w), p1_aff, p2_aff, nt, p1_relu)
    y = y + bb[None, None, :]
    aff = _bn_aff(y, sh3, gg, be)
    for (w, bb, gg, be) in params[1:]:
        y = _mm(y, _prep_w(w), aff, mt)
        y = y + bb[None, None, :]
        aff = _bn_aff(y, sh3, gg, be)
    return y, aff


def kernel(x, params):
    x = x.astype(jnp.float32)
    b, _, n = x.shape
    xyz_t = x
    xyz = jnp.transpose(x, (0, 2, 1))

    xyz1, xyz1_t, f1, aff1 = _sa_stage(
        xyz, xyz_t, None, None, 1024, 0.1, params['sa1'], 256, 6400)
    xyz2, xyz2_t, f2, aff2 = _sa_stage(
        xyz1, xyz1_t, f1, aff1, 256, 0.2, params['sa2'], 256, 6400)
    xyz3, xyz3_t, f3, aff3 = _sa_stage(
        xyz2, xyz2_t, f2, aff2, 64, 0.4, params['sa3'], 64, 3200)
    xyz4, xyz4_t, f4, aff4 = _sa_stage(
        xyz3, xyz3_t, f3, aff3, 16, 0.8, params['sa4'], 16, 800)

    l3, affl3 = _fp_stage(xyz3, xyz4, f3, aff3, True, f4, aff4,
                          params['fp1'], 64, 64)
    l2, affl2 = _fp_stage(xyz2, xyz3, f2, aff2, True, l3, affl3,
                          params['fp2'], 256, 256)
    l1, affl1 = _fp_stage(xyz1, xyz2, f1, aff1, True, l2, affl2,
                          params['fp3'], 1024, 1024)
    l0, affl0 = _fp_stage(xyz, xyz1, xyz, None, False, l1, affl1,
                          params['fp4'], 1024, 2048)

    (wh1, bh1, gh1, beh1) = params['head1'][0]
    yh = _mm(l0, _prep_w(wh1), affl0, 2048)
    yh = yh + bh1[None, None, :]
    affh = _bn_aff(yh, (b, n, -1), gh1, beh1)
    (wh2, bh2, _, _) = params['head2'][0]
    out = _head2(yh, _prep_w(wh2), bh2[None, :], affh, 2048)
    return out
